# Initial kernel scaffold; baseline (speedup 1.0000x reference)
import jax
import jax.numpy as jnp
from jax.experimental import pallas as pl


def kernel(features, coords):
    raise NotImplementedError("write your pallas kernel here")


# speedup vs baseline: 5.8889x; 5.8889x over previous
import jax
import jax.numpy as jnp
from jax import lax
from jax.experimental import pallas as pl

_S3 = 8
_C = 128


def _feat_body(f_ref, o_ref):
    b = f_ref.shape[0]
    o_ref[...] = jnp.broadcast_to(f_ref[...][:, None, :], (b, _S3, _C))


def _coord_body(c_ref, o_ref):
    b = c_ref.shape[0]
    jj = lax.broadcasted_iota(jnp.int32, (b, _S3, 3), 1)
    kk = lax.broadcasted_iota(jnp.int32, (b, _S3, 3), 2)
    off = lax.shift_right_logical(jj, 2 - kk) & 1
    o_ref[...] = c_ref[...][:, None, :] * 2 + off


def kernel(features, coords):
    n, c = features.shape
    bf = 1000
    fine3 = pl.pallas_call(
        _feat_body,
        grid=(n // bf,),
        in_specs=[pl.BlockSpec((bf, c), lambda i: (i, 0))],
        out_specs=pl.BlockSpec((bf, _S3, c), lambda i: (i, 0, 0)),
        out_shape=jax.ShapeDtypeStruct((n, _S3, c), jnp.float32),
    )(features)

    bc = 2000
    fine_c3 = pl.pallas_call(
        _coord_body,
        grid=(n // bc,),
        in_specs=[pl.BlockSpec((bc, 3), lambda i: (i, 0))],
        out_specs=pl.BlockSpec((bc, _S3, 3), lambda i: (i, 0, 0)),
        out_shape=jax.ShapeDtypeStruct((n, _S3, 3), jnp.int32),
    )(coords)

    return fine3.reshape(n * _S3, c), fine_c3.reshape(n * _S3, 3)
